# baseline (speedup 1.0000x reference)
import functools

import jax
import jax.numpy as jnp
from jax import lax
from jax.experimental import pallas as pl
from jax.experimental.pallas import tpu as pltpu
from jax.experimental.pallas import tpu_sc as plsc

_N = 100000
_D = 128
_G = 256
_B = 20000
_NB = _N // _B
_NW = 32
_RW = _N // _NW
_NV = -(-_RW // 16)
_RP = _NV * 16


def _gate_body(B, NB):
    def body(x_ref, wg_ref, gate_ref, gmax_ref, m_ref):
        i = pl.program_id(0)
        gateT = lax.dot_general(wg_ref[...], x_ref[...],
                                (((0,), (1,)), ((), ())),
                                preferred_element_type=jnp.float32)
        gate_ref[0] = gateT
        bm = jnp.max(gateT)

        @pl.when(i == 0)
        def _first():
            m_ref[0, 0] = bm

        @pl.when(i > 0)
        def _rest():
            m_ref[0, 0] = jnp.maximum(m_ref[0, 0], bm)

        @pl.when(i == NB - 1)
        def _fin():
            gmax_ref[...] = jnp.full((1, 128), m_ref[0, 0], jnp.float32)

    return body


def _gate_call(N, D, B):
    NB = N // B
    return pl.pallas_call(
        _gate_body(B, NB),
        grid=(NB,),
        in_specs=[
            pl.BlockSpec((B, D), lambda i: (i, 0)),
            pl.BlockSpec((D, 1), lambda i: (0, 0)),
        ],
        out_specs=[
            pl.BlockSpec((1, 1, B), lambda i: (i, 0, 0)),
            pl.BlockSpec((1, 128), lambda i: (0, 0)),
        ],
        out_shape=[
            jax.ShapeDtypeStruct((NB, 1, B), jnp.float32),
            jax.ShapeDtypeStruct((1, 128), jnp.float32),
        ],
        scratch_shapes=[pltpu.SMEM((1, 1), jnp.float32)],
    )


def _sc_mesh():
    return plsc.VectorSubcoreMesh(core_axis_name="c", subcore_axis_name="s")


def _make_denom_kernel():
    @functools.partial(
        pl.kernel,
        mesh=_sc_mesh(),
        out_type=jax.ShapeDtypeStruct((_NW, _G), jnp.float32),
        compiler_params=pltpu.CompilerParams(needs_layout_passes=False),
        scratch_types=[
            pltpu.VMEM((_RP,), jnp.float32),
            pltpu.VMEM((_RP,), jnp.int32),
            pltpu.VMEM((_G,), jnp.float32),
            pltpu.VMEM((16,), jnp.float32),
        ],
    )
    def denom(gate_hbm, seg_hbm, gmax_hbm, dpart_hbm, gbuf, sbuf, dloc, gv):
        wid = lax.axis_index("s") * 2 + lax.axis_index("c")
        pltpu.sync_copy(gate_hbm.at[wid], gbuf)
        pltpu.sync_copy(seg_hbm.at[wid], sbuf)
        pltpu.sync_copy(gmax_hbm.at[pl.ds(0, 16)], gv)
        for c in range(_G // 16):
            dloc[pl.ds(c * 16, 16)] = jnp.zeros((16,), jnp.float32)
        gmaxv = gv[...]

        def body(j, carry):
            off = j * 16
            e = jnp.exp(gbuf[pl.ds(off, 16)] - gmaxv)
            sg = sbuf[pl.ds(off, 16)]
            msk = (lax.iota(jnp.int32, 16) + off) < _RW
            plsc.addupdate_scatter(dloc, [sg], e, mask=msk)
            return carry

        lax.fori_loop(0, _NV, body, 0)
        pltpu.sync_copy(dloc, dpart_hbm.at[wid])

    return denom


def _make_alpha_kernel():
    @functools.partial(
        pl.kernel,
        mesh=_sc_mesh(),
        out_type=jax.ShapeDtypeStruct((_NW, _RP), jnp.float32),
        compiler_params=pltpu.CompilerParams(needs_layout_passes=False),
        scratch_types=[
            pltpu.VMEM((_RP,), jnp.float32),
            pltpu.VMEM((_RP,), jnp.int32),
            pltpu.VMEM((_RP,), jnp.float32),
            pltpu.VMEM((_NW, _G), jnp.float32),
            pltpu.VMEM((_G,), jnp.float32),
            pltpu.VMEM((16,), jnp.float32),
        ],
    )
    def alpha(gate_hbm, seg_hbm, gmax_hbm, dpart_hbm, alpha_hbm,
              gbuf, sbuf, abuf, dp, dsum, gv):
        wid = lax.axis_index("s") * 2 + lax.axis_index("c")
        pltpu.sync_copy(gate_hbm.at[wid], gbuf)
        pltpu.sync_copy(seg_hbm.at[wid], sbuf)
        pltpu.sync_copy(gmax_hbm.at[pl.ds(0, 16)], gv)
        pltpu.sync_copy(dpart_hbm, dp)
        gmaxv = gv[...]
        for c in range(_G // 16):
            acc = dp[0, pl.ds(c * 16, 16)]
            for r in range(1, _NW):
                acc = acc + dp[r, pl.ds(c * 16, 16)]
            dsum[pl.ds(c * 16, 16)] = acc

        def body(j, carry):
            off = j * 16
            e = jnp.exp(gbuf[pl.ds(off, 16)] - gmaxv)
            sg = sbuf[pl.ds(off, 16)]
            dg = plsc.load_gather(dsum, [sg])
            abuf[pl.ds(off, 16)] = e / dg
            return carry

        lax.fori_loop(0, _NV, body, 0)
        pltpu.sync_copy(abuf, alpha_hbm.at[wid])

    return alpha


def _pool_body(B, G, D, NB):
    def body(x_ref, seg_ref, a_ref, out_ref, acc_ref):
        i = pl.program_id(0)

        @pl.when(i == 0)
        def _init():
            acc_ref[...] = jnp.zeros((G, D), jnp.float32)

        s = seg_ref[0]
        aT = a_ref[0]
        rows = lax.broadcasted_iota(jnp.int32, (G, B), 0)
        maskf = (rows == s).astype(jnp.float32)
        me = (maskf * aT).astype(jnp.bfloat16)
        contrib = lax.dot_general(me, x_ref[...].astype(jnp.bfloat16),
                                  (((1,), (0,)), ((), ())),
                                  preferred_element_type=jnp.float32)
        acc_ref[...] = acc_ref[...] + contrib

        @pl.when(i == NB - 1)
        def _fin():
            out_ref[...] = acc_ref[...]

    return body


def _pool_call(N, D, G, B):
    NB = N // B
    return pl.pallas_call(
        _pool_body(B, G, D, NB),
        grid=(NB,),
        in_specs=[
            pl.BlockSpec((B, D), lambda i: (i, 0)),
            pl.BlockSpec((1, 1, B), lambda i: (i, 0, 0)),
            pl.BlockSpec((1, 1, B), lambda i: (i, 0, 0)),
        ],
        out_specs=pl.BlockSpec((G, D), lambda i: (0, 0)),
        out_shape=jax.ShapeDtypeStruct((G, D), jnp.float32),
        scratch_shapes=[pltpu.VMEM((G, D), jnp.float32)],
    )


def kernel(features, segment_ids, Wg, bg):
    N, D = features.shape
    seg = segment_ids.astype(jnp.int32)

    gate, gmax = _gate_call(N, D, _B)(features, Wg)

    pad = ((0, 0), (0, _RP - _RW))
    gate_w = jnp.pad(gate.reshape(_NW, _RW), pad)
    seg_w = jnp.pad(seg.reshape(_NW, _RW), pad)
    gmax_v = gmax.reshape(128)

    dpart = _make_denom_kernel()(gate_w, seg_w, gmax_v)
    alpha_p = _make_alpha_kernel()(gate_w, seg_w, gmax_v, dpart)

    alpha = alpha_p[:, :_RW].reshape(_NB, 1, _B)
    return _pool_call(N, D, _G, _B)(features, seg.reshape(_NB, 1, _B), alpha)


# speedup vs baseline: 1.1437x; 1.1437x over previous
import functools

import jax
import jax.numpy as jnp
from jax import lax
from jax.experimental import pallas as pl
from jax.experimental.pallas import tpu as pltpu
from jax.experimental.pallas import tpu_sc as plsc

_N = 100000
_D = 128
_G = 256
_B = 20000
_NB = _N // _B
_NW = 32
_RW = _N // _NW
_NV = -(-_RW // 16)
_RP = _NV * 16


def _gate_body(B, NB):
    def body(x_ref, wg_ref, gate_ref, gmax_ref, m_ref):
        i = pl.program_id(0)
        gateT = lax.dot_general(wg_ref[...], x_ref[...],
                                (((0,), (1,)), ((), ())),
                                preferred_element_type=jnp.float32)
        gate_ref[0] = gateT
        bm = jnp.max(gateT)

        @pl.when(i == 0)
        def _first():
            m_ref[0, 0] = bm

        @pl.when(i > 0)
        def _rest():
            m_ref[0, 0] = jnp.maximum(m_ref[0, 0], bm)

        @pl.when(i == NB - 1)
        def _fin():
            gmax_ref[...] = jnp.full((1, 128), m_ref[0, 0], jnp.float32)

    return body


def _gate_call(N, D, B):
    NB = N // B
    return pl.pallas_call(
        _gate_body(B, NB),
        grid=(NB,),
        in_specs=[
            pl.BlockSpec((B, D), lambda i: (i, 0)),
            pl.BlockSpec((D, 1), lambda i: (0, 0)),
        ],
        out_specs=[
            pl.BlockSpec((1, 1, B), lambda i: (i, 0, 0)),
            pl.BlockSpec((1, 128), lambda i: (0, 0)),
        ],
        out_shape=[
            jax.ShapeDtypeStruct((NB, 1, B), jnp.float32),
            jax.ShapeDtypeStruct((1, 128), jnp.float32),
        ],
        scratch_shapes=[pltpu.SMEM((1, 1), jnp.float32)],
    )


def _sc_mesh():
    return plsc.VectorSubcoreMesh(core_axis_name="c", subcore_axis_name="s")


def _make_denom_kernel():
    @functools.partial(
        pl.kernel,
        mesh=_sc_mesh(),
        out_type=jax.ShapeDtypeStruct((_NW, _G), jnp.float32),
        compiler_params=pltpu.CompilerParams(needs_layout_passes=False),
        scratch_types=[
            pltpu.VMEM((_RP,), jnp.float32),
            pltpu.VMEM((_RP,), jnp.int32),
            pltpu.VMEM((_G,), jnp.float32),
            pltpu.VMEM((16,), jnp.float32),
        ],
    )
    def denom(gate_hbm, seg_hbm, gmax_hbm, dpart_hbm, gbuf, sbuf, dloc, gv):
        wid = lax.axis_index("s") * 2 + lax.axis_index("c")
        pltpu.sync_copy(gate_hbm.at[wid], gbuf)
        pltpu.sync_copy(seg_hbm.at[wid], sbuf)
        pltpu.sync_copy(gmax_hbm.at[pl.ds(0, 16)], gv)
        for c in range(_G // 16):
            dloc[pl.ds(c * 16, 16)] = jnp.zeros((16,), jnp.float32)
        gmaxv = gv[...]

        def body(j, carry):
            off = j * 16
            e = jnp.exp(gbuf[pl.ds(off, 16)] - gmaxv)
            sg = sbuf[pl.ds(off, 16)]
            msk = (lax.iota(jnp.int32, 16) + off) < _RW
            plsc.addupdate_scatter(dloc, [sg], e, mask=msk)
            return carry

        lax.fori_loop(0, _NV, body, 0)
        pltpu.sync_copy(dloc, dpart_hbm.at[wid])

    return denom


def _pool_body(B, G, D, NB, NW):
    def body(x_ref, seg_ref, wg_ref, gmax_ref, dpart_ref, out_ref, acc_ref):
        i = pl.program_id(0)

        @pl.when(i == 0)
        def _init():
            acc_ref[...] = jnp.zeros((G, D), jnp.float32)

        x = x_ref[...]
        s = seg_ref[0]
        gateT = lax.dot_general(wg_ref[...], x, (((0,), (1,)), ((), ())),
                                preferred_element_type=jnp.float32)
        eT = jnp.exp(gateT - gmax_ref[0, 0])
        rows = lax.broadcasted_iota(jnp.int32, (G, B), 0)
        maskf = (rows == s).astype(jnp.float32)
        me = (maskf * eT).astype(jnp.bfloat16)
        contrib = lax.dot_general(me, x.astype(jnp.bfloat16),
                                  (((1,), (0,)), ((), ())),
                                  preferred_element_type=jnp.float32)
        acc_ref[...] = acc_ref[...] + contrib

        @pl.when(i == NB - 1)
        def _fin():
            ones_w = jnp.ones((NW, 1), jnp.float32)
            d = lax.dot_general(dpart_ref[...], ones_w,
                                (((0,), (0,)), ((), ())),
                                preferred_element_type=jnp.float32)
            recip = jnp.where(d > 0, jnp.float32(1.0) / d, jnp.float32(0.0))
            out_ref[...] = acc_ref[...] * recip

    return body


def _pool_call(N, D, G, B, NW):
    NB = N // B
    return pl.pallas_call(
        _pool_body(B, G, D, NB, NW),
        grid=(NB,),
        in_specs=[
            pl.BlockSpec((B, D), lambda i: (i, 0)),
            pl.BlockSpec((1, 1, B), lambda i: (i, 0, 0)),
            pl.BlockSpec((D, 1), lambda i: (0, 0)),
            pl.BlockSpec((1, 128), lambda i: (0, 0)),
            pl.BlockSpec((NW, G), lambda i: (0, 0)),
        ],
        out_specs=pl.BlockSpec((G, D), lambda i: (0, 0)),
        out_shape=jax.ShapeDtypeStruct((G, D), jnp.float32),
        scratch_shapes=[pltpu.VMEM((G, D), jnp.float32)],
    )


def kernel(features, segment_ids, Wg, bg):
    N, D = features.shape
    seg = segment_ids.astype(jnp.int32)

    gate, gmax = _gate_call(N, D, _B)(features, Wg)

    pad = ((0, 0), (0, _RP - _RW))
    gate_w = jnp.pad(gate.reshape(_NW, _RW), pad)
    seg_w = jnp.pad(seg.reshape(_NW, _RW), pad)
    gmax_v = gmax.reshape(128)

    dpart = _make_denom_kernel()(gate_w, seg_w, gmax_v)

    return _pool_call(N, D, _G, _B, _NW)(
        features, seg.reshape(_NB, 1, _B), Wg, gmax, dpart)
